# baseline (speedup 1.0000x reference)
import functools

import jax
import jax.numpy as jnp
from jax import lax
from jax.experimental import pallas as pl
from jax.experimental.pallas import tpu as pltpu
from jax.experimental.pallas import tpu_sc as plsc

V = 1000000
D = 64
S = 50
S_PAD = 56
NC, NS = 2, 16
NW = NC * NS

BG = 256
N_SB = 999936 // BG
N_LAP = 125
V_TAIL = 999936


def _make_transpose():
    mesh = plsc.VectorSubcoreMesh(core_axis_name="c", subcore_axis_name="s")

    @functools.partial(
        pl.kernel,
        out_type=jax.ShapeDtypeStruct((V, 128), jnp.float32),
        mesh=mesh,
        scratch_types=[
            pltpu.VMEM((2, D, BG), jnp.float32),
            pltpu.VMEM((2, BG, 128), jnp.float32),
            pltpu.VMEM((D, 64), jnp.float32),
            pltpu.VMEM((64, 128), jnp.float32),
            pltpu.SemaphoreType.DMA,
            pltpu.SemaphoreType.DMA,
            pltpu.SemaphoreType.DMA,
            pltpu.SemaphoreType.DMA,
        ],
        compiler_params=pltpu.CompilerParams(use_tc_tiling_on_sc=True,
                                             needs_layout_passes=False),
    )
    def transpose_kernel(wt_hbm, out_hbm, blk_v, sbuf_v, tblk_v, tsb_v,
                         sr0, sr1, sw0, sw1):
        s_r = (sr0, sr1)
        s_w = (sw0, sw1)
        wid = lax.axis_index("s") * NC + lax.axis_index("c")
        iota = lax.iota(jnp.int32, 16)

        def sb_of(i):
            return jnp.minimum(wid * N_LAP + i, N_SB - 1)

        def start_r(i, b):
            off = pl.multiple_of(sb_of(i) * BG, 8)
            pltpu.async_copy(wt_hbm.at[:, pl.ds(off, BG)], blk_v.at[b], s_r[b])

        def wait_r(b):
            pltpu.make_async_copy(wt_hbm.at[:, pl.ds(0, BG)],
                                  blk_v.at[b], s_r[b]).wait()

        def start_w(i, b):
            off = pl.multiple_of(sb_of(i) * BG, 8)
            pltpu.async_copy(sbuf_v.at[b], out_hbm.at[pl.ds(off, BG), :],
                             s_w[b])

        def wait_w(b):
            pltpu.make_async_copy(sbuf_v.at[b],
                                  out_hbm.at[pl.ds(0, BG), :], s_w[b]).wait()

        def transpose_block(b):
            def dbody(d, carry):
                dv = jnp.full((16,), 0, jnp.int32) + d
                for jg in range(BG // 16):
                    jv = iota + (jg * 16)
                    x = blk_v[b, d, pl.ds(jg * 16, 16)]
                    plsc.store_scatter(sbuf_v.at[b], [jv, dv], x)
                return carry

            lax.fori_loop(0, D, dbody, 0)

        start_r(0, 0)
        wait_r(0)
        start_r(1, 1)
        transpose_block(0)
        start_w(0, 0)
        wait_r(1)
        start_r(2, 0)
        transpose_block(1)
        start_w(1, 1)

        def lap(i, b):
            nb = 1 - b
            wait_r(b)
            start_r(i + 1, nb)
            wait_w(b)
            transpose_block(b)
            start_w(i, b)

        def body(j, carry):
            lap(2 + 2 * j, 0)
            lap(3 + 2 * j, 1)
            return carry

        lax.fori_loop(0, (N_LAP - 3) // 2, body, 0)

        wait_r(0)
        wait_w(0)
        transpose_block(0)
        start_w(N_LAP - 1, 0)
        wait_w(1)
        wait_w(0)

        @pl.when(wid == 0)
        def _():
            pltpu.sync_copy(wt_hbm.at[:, pl.ds(V_TAIL, 64)], tblk_v)

            def dbody(d, carry):
                dv = jnp.full((16,), 0, jnp.int32) + d
                for jg in range(4):
                    jv = iota + (jg * 16)
                    x = tblk_v[d, pl.ds(jg * 16, 16)]
                    plsc.store_scatter(tsb_v, [jv, dv], x)
                return carry

            lax.fori_loop(0, D, dbody, 0)
            pltpu.sync_copy(tsb_v, out_hbm.at[pl.ds(V_TAIL, 64), :])

    return transpose_kernel


def _make_gather(n_b, nb_per_grp):
    b_per_w = n_b // NW
    ngrp = b_per_w // nb_per_grp
    C = nb_per_grp * S
    assert ngrp % 2 == 0 and ngrp >= 4
    mesh = plsc.VectorSubcoreMesh(core_axis_name="c", subcore_axis_name="s")

    @functools.partial(
        pl.kernel,
        out_type=jax.ShapeDtypeStruct((n_b, S_PAD, 128), jnp.float32),
        mesh=mesh,
        scratch_types=[
            pltpu.VMEM((2, C), jnp.int32),
            pltpu.VMEM((2, C, 128), jnp.float32),
            pltpu.SemaphoreType.DMA,
            pltpu.SemaphoreType.DMA,
            pltpu.SemaphoreType.DMA,
            pltpu.SemaphoreType.DMA,
            pltpu.SemaphoreType.DMA,
            pltpu.SemaphoreType.DMA,
        ],
        compiler_params=pltpu.CompilerParams(use_tc_tiling_on_sc=False),
    )
    def gather_kernel(table_hbm, idx_hbm, out_hbm, idx_v, rows_v,
                      si0, si1, sg0, sg1, sw0, sw1):
        s_idx = (si0, si1)
        s_g = (sg0, sg1)
        s_w = (sw0, sw1)
        wid = lax.axis_index("s") * NC + lax.axis_index("c")
        base_b = wid * b_per_w

        def idx_off(k):
            return pl.multiple_of(
                jnp.minimum((base_b + k * nb_per_grp) * S,
                            (n_b - nb_per_grp) * S), 8)

        def start_idx(k, b):
            pltpu.async_copy(idx_hbm.at[pl.ds(idx_off(k), C)],
                             idx_v.at[b], s_idx[b])

        def wait_idx(b):
            pltpu.make_async_copy(idx_hbm.at[pl.ds(0, C)],
                                  idx_v.at[b], s_idx[b]).wait()

        def start_g(b):
            pltpu.async_copy(table_hbm.at[idx_v.at[b]], rows_v.at[b], s_g[b])

        def wait_g(b):
            pltpu.make_async_copy(table_hbm.at[pl.ds(0, C)],
                                  rows_v.at[b], s_g[b]).wait()

        def start_w(k, b):
            b0 = base_b + k * nb_per_grp
            for j in range(nb_per_grp):
                pltpu.async_copy(
                    rows_v.at[b, pl.ds(j * S, S), pl.ds(0, D)],
                    out_hbm.at[b0 + j, pl.ds(0, S), pl.ds(0, D)],
                    s_w[b])

        def wait_w(b):
            for _ in range(nb_per_grp):
                pltpu.make_async_copy(
                    rows_v.at[b, pl.ds(0, S), pl.ds(0, D)],
                    out_hbm.at[0, pl.ds(0, S), pl.ds(0, D)],
                    s_w[b]).wait()

        start_idx(0, 0)
        wait_idx(0)
        start_g(0)
        start_idx(1, 1)
        wait_g(0)
        wait_idx(1)
        start_g(1)
        start_w(0, 0)
        start_idx(2, 0)

        def lap(k, b):
            nb = 1 - b
            wait_g(b)
            wait_idx(nb)
            wait_w(nb)
            start_g(nb)
            start_w(k, b)
            start_idx(k + 2, b)

        def body(j, carry):
            lap(1 + 2 * j, 1)
            lap(2 + 2 * j, 0)
            return carry

        lax.fori_loop(0, (ngrp - 2) // 2, body, 0)

        wait_g(1)
        start_w(ngrp - 1, 1)
        wait_w(0)
        wait_w(1)
        wait_idx(0)

    return gather_kernel


def kernel(weight, token_ids):
    B0, S0 = token_ids.shape
    flat = token_ids.reshape(B0 * S0)
    tslot = _make_transpose()(weight.T)
    out_pad = _make_gather(B0, 8)(tslot, flat)
    return out_pad[:, :S0, :D]


# speedup vs baseline: 1.2716x; 1.2716x over previous
import functools

import jax
import jax.numpy as jnp
from jax import lax
from jax.experimental import pallas as pl
from jax.experimental.pallas import tpu as pltpu
from jax.experimental.pallas import tpu_sc as plsc

V = 1000000
D = 64
S = 50
S_PAD = 56
NC, NS = 2, 16
NW = NC * NS

BG = 256
N_SB = 999936 // BG
N_LAP = 125
V_TAIL = 999936


def _make_transpose():
    mesh = plsc.VectorSubcoreMesh(core_axis_name="c", subcore_axis_name="s")

    @functools.partial(
        pl.kernel,
        out_type=jax.ShapeDtypeStruct((V, 128), jnp.float32),
        mesh=mesh,
        scratch_types=[
            pltpu.VMEM((2, D, BG), jnp.float32),
            pltpu.VMEM((2, BG, 128), jnp.float32),
            pltpu.VMEM((D, 64), jnp.float32),
            pltpu.VMEM((64, 128), jnp.float32),
            pltpu.SemaphoreType.DMA,
            pltpu.SemaphoreType.DMA,
            pltpu.SemaphoreType.DMA,
            pltpu.SemaphoreType.DMA,
        ],
        compiler_params=pltpu.CompilerParams(use_tc_tiling_on_sc=True,
                                             needs_layout_passes=False),
    )
    def transpose_kernel(wt_hbm, out_hbm, blk_v, sbuf_v, tblk_v, tsb_v,
                         sr0, sr1, sw0, sw1):
        s_r = (sr0, sr1)
        s_w = (sw0, sw1)
        wid = lax.axis_index("s") * NC + lax.axis_index("c")
        iota = lax.iota(jnp.int32, 16)
        jvs = [iota + (jg * 16) for jg in range(BG // 16)]

        def sb_of(i):
            return jnp.minimum(wid * N_LAP + i, N_SB - 1)

        def start_r(i, b):
            off = pl.multiple_of(sb_of(i) * BG, 8)
            pltpu.async_copy(wt_hbm.at[:, pl.ds(off, BG)], blk_v.at[b], s_r[b])

        def wait_r(b):
            pltpu.make_async_copy(wt_hbm.at[:, pl.ds(0, BG)],
                                  blk_v.at[b], s_r[b]).wait()

        def start_w(i, b):
            off = pl.multiple_of(sb_of(i) * BG, 8)
            pltpu.async_copy(sbuf_v.at[b], out_hbm.at[pl.ds(off, BG), :],
                             s_w[b])

        def wait_w(b):
            pltpu.make_async_copy(sbuf_v.at[b],
                                  out_hbm.at[pl.ds(0, BG), :], s_w[b]).wait()

        def transpose_block(b):
            @plsc.parallel_loop(0, D, unroll=4)
            def _(d):
                dv = jnp.full((16,), 0, jnp.int32) + d
                for jg in range(BG // 16):
                    x = blk_v[b, d, pl.ds(jg * 16, 16)]
                    plsc.store_scatter(sbuf_v.at[b], [jvs[jg], dv], x)

        start_r(0, 0)
        wait_r(0)
        start_r(1, 1)
        transpose_block(0)
        start_w(0, 0)
        wait_r(1)
        start_r(2, 0)
        transpose_block(1)
        start_w(1, 1)

        def lap(i, b):
            nb = 1 - b
            wait_r(b)
            start_r(i + 1, nb)
            wait_w(b)
            transpose_block(b)
            start_w(i, b)

        def body(j, carry):
            lap(2 + 2 * j, 0)
            lap(3 + 2 * j, 1)
            return carry

        lax.fori_loop(0, (N_LAP - 3) // 2, body, 0)

        wait_r(0)
        wait_w(0)
        transpose_block(0)
        start_w(N_LAP - 1, 0)
        wait_w(1)
        wait_w(0)

        @pl.when(wid == 0)
        def _():
            pltpu.sync_copy(wt_hbm.at[:, pl.ds(V_TAIL, 64)], tblk_v)

            @plsc.parallel_loop(0, D, unroll=4)
            def _(d):
                dv = jnp.full((16,), 0, jnp.int32) + d
                for jg in range(4):
                    x = tblk_v[d, pl.ds(jg * 16, 16)]
                    plsc.store_scatter(tsb_v, [jvs[jg], dv], x)

            pltpu.sync_copy(tsb_v, out_hbm.at[pl.ds(V_TAIL, 64), :])

    return transpose_kernel


def _make_gather(n_b, nb_per_grp):
    b_per_w = n_b // NW
    ngrp = b_per_w // nb_per_grp
    C = nb_per_grp * S
    assert ngrp % 2 == 0 and ngrp >= 4
    mesh = plsc.VectorSubcoreMesh(core_axis_name="c", subcore_axis_name="s")

    @functools.partial(
        pl.kernel,
        out_type=jax.ShapeDtypeStruct((n_b, S_PAD, 128), jnp.float32),
        mesh=mesh,
        scratch_types=[
            pltpu.VMEM((2, C), jnp.int32),
            pltpu.VMEM((2, C, 128), jnp.float32),
            pltpu.SemaphoreType.DMA,
            pltpu.SemaphoreType.DMA,
            pltpu.SemaphoreType.DMA,
            pltpu.SemaphoreType.DMA,
            pltpu.SemaphoreType.DMA,
            pltpu.SemaphoreType.DMA,
        ],
        compiler_params=pltpu.CompilerParams(use_tc_tiling_on_sc=False),
    )
    def gather_kernel(table_hbm, idx_hbm, out_hbm, idx_v, rows_v,
                      si0, si1, sg0, sg1, sw0, sw1):
        s_idx = (si0, si1)
        s_g = (sg0, sg1)
        s_w = (sw0, sw1)
        wid = lax.axis_index("s") * NC + lax.axis_index("c")
        base_b = wid * b_per_w

        def idx_off(k):
            return pl.multiple_of(
                jnp.minimum((base_b + k * nb_per_grp) * S, (n_b - nb_per_grp) * S), 8)

        def start_idx(k, b):
            pltpu.async_copy(idx_hbm.at[pl.ds(idx_off(k), C)],
                             idx_v.at[b], s_idx[b])

        def wait_idx(b):
            pltpu.make_async_copy(idx_hbm.at[pl.ds(0, C)],
                                  idx_v.at[b], s_idx[b]).wait()

        def start_g(b):
            pltpu.async_copy(table_hbm.at[idx_v.at[b]], rows_v.at[b], s_g[b])

        def wait_g(b):
            pltpu.make_async_copy(table_hbm.at[pl.ds(0, C)],
                                  rows_v.at[b], s_g[b]).wait()

        def start_w(k, b):
            b0 = base_b + k * nb_per_grp
            for j in range(nb_per_grp):
                pltpu.async_copy(
                    rows_v.at[b, pl.ds(j * S, S), pl.ds(0, D)],
                    out_hbm.at[b0 + j, pl.ds(0, S), pl.ds(0, D)],
                    s_w[b])

        def wait_w(b):
            for _ in range(nb_per_grp):
                pltpu.make_async_copy(
                    rows_v.at[b, pl.ds(0, S), pl.ds(0, D)],
                    out_hbm.at[0, pl.ds(0, S), pl.ds(0, D)],
                    s_w[b]).wait()

        start_idx(0, 0)
        wait_idx(0)
        start_g(0)
        start_idx(1, 1)
        wait_g(0)
        wait_idx(1)
        start_g(1)
        start_w(0, 0)
        start_idx(2, 0)

        def lap(k, b):
            nb = 1 - b
            wait_g(b)
            wait_idx(nb)
            wait_w(nb)
            start_g(nb)
            start_w(k, b)
            start_idx(k + 2, b)

        def body(j, carry):
            lap(1 + 2 * j, 1)
            lap(2 + 2 * j, 0)
            return carry

        lax.fori_loop(0, (ngrp - 2) // 2, body, 0)

        wait_g(1)
        start_w(ngrp - 1, 1)
        wait_w(0)
        wait_w(1)
        wait_idx(0)

    return gather_kernel


def kernel(weight, token_ids):
    B0, S0 = token_ids.shape
    flat = token_ids.reshape(B0 * S0)
    tslot = _make_transpose()(weight.T)
    out_pad = _make_gather(B0, 8)(tslot, flat)
    return out_pad[:, :S0, :D]


# speedup vs baseline: 1.7837x; 1.4027x over previous
import functools

import jax
import jax.numpy as jnp
from jax import lax
from jax.experimental import pallas as pl
from jax.experimental.pallas import tpu as pltpu
from jax.experimental.pallas import tpu_sc as plsc

D = 64
S = 50
S_PAD = 56
NC, NS = 2, 16
NW = NC * NS


def _make_gather(n_b, nb_per_grp):
    b_per_w = n_b // NW
    ngrp = b_per_w // nb_per_grp
    C = nb_per_grp * S
    assert ngrp % 2 == 0 and ngrp >= 4
    mesh = plsc.VectorSubcoreMesh(core_axis_name="c", subcore_axis_name="s")

    @functools.partial(
        pl.kernel,
        out_type=jax.ShapeDtypeStruct((n_b, S_PAD, 128), jnp.float32),
        mesh=mesh,
        scratch_types=[
            pltpu.VMEM((2, C), jnp.int32),
            pltpu.VMEM((2, C, D), jnp.float32),
            pltpu.SemaphoreType.DMA,
            pltpu.SemaphoreType.DMA,
            pltpu.SemaphoreType.DMA,
            pltpu.SemaphoreType.DMA,
            pltpu.SemaphoreType.DMA,
            pltpu.SemaphoreType.DMA,
        ],
        compiler_params=pltpu.CompilerParams(use_tc_tiling_on_sc=False),
    )
    def gather_kernel(table_hbm, idx_hbm, out_hbm, idx_v, rows_v,
                      si0, si1, sg0, sg1, sw0, sw1):
        s_idx = (si0, si1)
        s_g = (sg0, sg1)
        s_w = (sw0, sw1)
        wid = lax.axis_index("s") * NC + lax.axis_index("c")
        base_b = wid * b_per_w

        def idx_off(k):
            return pl.multiple_of(
                jnp.minimum((base_b + k * nb_per_grp) * S, (n_b - nb_per_grp) * S), 8)

        def start_idx(k, b):
            pltpu.async_copy(idx_hbm.at[pl.ds(idx_off(k), C)],
                             idx_v.at[b], s_idx[b])

        def wait_idx(b):
            pltpu.make_async_copy(idx_hbm.at[pl.ds(0, C)],
                                  idx_v.at[b], s_idx[b]).wait()

        def start_g(b):
            pltpu.async_copy(table_hbm.at[idx_v.at[b]], rows_v.at[b], s_g[b])

        def wait_g(b):
            pltpu.make_async_copy(table_hbm.at[pl.ds(0, C)],
                                  rows_v.at[b], s_g[b]).wait()

        def start_w(k, b):
            b0 = base_b + k * nb_per_grp
            for j in range(nb_per_grp):
                pltpu.async_copy(
                    rows_v.at[b, pl.ds(j * S, S), :],
                    out_hbm.at[b0 + j, pl.ds(0, S), pl.ds(0, D)],
                    s_w[b])

        def wait_w(b):
            for _ in range(nb_per_grp):
                pltpu.make_async_copy(
                    rows_v.at[b, pl.ds(0, S), :],
                    out_hbm.at[0, pl.ds(0, S), pl.ds(0, D)],
                    s_w[b]).wait()

        start_idx(0, 0)
        wait_idx(0)
        start_g(0)
        start_idx(1, 1)
        wait_g(0)
        wait_idx(1)
        start_g(1)
        start_w(0, 0)
        start_idx(2, 0)

        def lap(k, b):
            nb = 1 - b
            wait_g(b)
            wait_idx(nb)
            wait_w(nb)
            start_g(nb)
            start_w(k, b)
            start_idx(k + 2, b)

        def body(j, carry):
            lap(1 + 2 * j, 1)
            lap(2 + 2 * j, 0)
            return carry

        lax.fori_loop(0, (ngrp - 2) // 2, body, 0)

        wait_g(1)
        start_w(ngrp - 1, 1)
        wait_w(0)
        wait_w(1)
        wait_idx(0)

    return gather_kernel


def kernel(weight, token_ids):
    B0, S0 = token_ids.shape
    flat = token_ids.reshape(B0 * S0)
    out_pad = _make_gather(B0, 16)(weight, flat)
    return out_pad[:, :S0, :D]
